# baseline (speedup 1.0000x reference)
import jax
import jax.numpy as jnp
from jax import lax
from jax.experimental import pallas as pl
from jax.experimental.pallas import tpu as pltpu
from jax.experimental.pallas import tpu_sc as plsc

N = 10000
NP = 10240
D = 128
E = 320000
NC = 2
NS = 16
NW = NC * NS
CHUNK = 48
SCH = 24
NSB = 9
EPT = CHUNK * SCH * NSB
E_PAD = NW * EPT
NBUF = 4
RPT = NP // NS

_LANES = 16


def _sc_body(x_hbm, col_hbm, row_hbm, adj_hbm, zero_hbm, out_hbm,
             colv, rowv, adjv, b0, b1, b2, b3, acc,
             g0, g1, g2, g3, s0, s1, s2, s3):
    bufs = (b0, b1, b2, b3)
    gsems = (g0, g1, g2, g3)
    ssems = (s0, s1, s2, s3)
    c = lax.axis_index("c")
    s = lax.axis_index("s")
    wid = s * NC + c

    pltpu.sync_copy(zero_hbm, acc.at[pl.ds(s * RPT, RPT)])
    plsc.subcore_barrier()

    def gather_start(k, b):
        pltpu.async_copy(x_hbm.at[colv.at[k]], bufs[b], gsems[b])

    def gather_wait(b):
        pltpu.make_async_copy(x_hbm.at[colv.at[0]], bufs[b], gsems[b]).wait()

    def scat_start(k, b):
        pltpu.async_copy(bufs[b], acc.at[rowv.at[k]], ssems[b], add=True)

    def scat_wait(b):
        pltpu.make_async_copy(bufs[b], acc.at[rowv.at[0]], ssems[b]).wait()

    def scale_chunk(k, b):
        rows = bufs[b]
        base = k * CHUNK
        for g in range(CHUNK // _LANES):
            a16 = adjv[pl.ds(base + g * _LANES, _LANES)]
            for j in range(_LANES):
                e = g * _LANES + j
                scale = jnp.full((_LANES,), a16[j], jnp.float32)
                for v in range(D // _LANES):
                    sl = pl.ds(v * _LANES, _LANES)
                    rows[e, sl] = rows[e, sl] * scale

    def sb_body(sb, carry):
        pltpu.sync_copy(col_hbm.at[wid, sb], colv)
        pltpu.sync_copy(row_hbm.at[wid, sb], rowv)
        pltpu.sync_copy(adj_hbm.at[wid, sb], adjv)

        for b in range(NBUF - 1):
            @pl.when(sb > 0)
            def _():
                scat_wait(b)
            gather_start(b, b)

        def quad_body(q, carry2):
            for b in range(NBUF):
                k = NBUF * q + b
                gather_wait(b)
                kk = k + NBUF - 1
                bb = (b + NBUF - 1) % NBUF
                if b == 0:
                    first_ever = jnp.logical_and(sb == 0, q == 0)

                    @pl.when(jnp.logical_and(kk < SCH,
                                             jnp.logical_not(first_ever)))
                    def _():
                        scat_wait(bb)
                        gather_start(kk, bb)

                    @pl.when(jnp.logical_and(kk < SCH, first_ever))
                    def _():
                        gather_start(kk, bb)
                else:
                    @pl.when(kk < SCH)
                    def _():
                        scat_wait(bb)
                        gather_start(kk, bb)
                scale_chunk(k, b)
                scat_start(k, b)
            return carry2

        lax.fori_loop(0, SCH // NBUF, quad_body, 0)
        return carry

    lax.fori_loop(0, NSB, sb_body, 0)
    for b in range(NBUF):
        scat_wait(b)
    plsc.subcore_barrier()
    pltpu.sync_copy(acc.at[pl.ds(s * RPT, RPT)],
                    out_hbm.at[c, pl.ds(s * RPT, RPT)])


_sc_aggregate = pl.kernel(
    _sc_body,
    out_type=jax.ShapeDtypeStruct((NC, NP, D), jnp.float32),
    mesh=plsc.VectorSubcoreMesh(
        core_axis_name="c", subcore_axis_name="s",
        num_cores=NC, num_subcores=NS),
    scratch_types=[
        pltpu.VMEM((SCH, CHUNK), jnp.int32),
        pltpu.VMEM((SCH, CHUNK), jnp.int32),
        pltpu.VMEM((SCH * CHUNK,), jnp.float32),
        pltpu.VMEM((CHUNK, D), jnp.float32),
        pltpu.VMEM((CHUNK, D), jnp.float32),
        pltpu.VMEM((CHUNK, D), jnp.float32),
        pltpu.VMEM((CHUNK, D), jnp.float32),
        pltpu.VMEM_SHARED((NP, D), jnp.float32),
        pltpu.SemaphoreType.DMA,
        pltpu.SemaphoreType.DMA,
        pltpu.SemaphoreType.DMA,
        pltpu.SemaphoreType.DMA,
        pltpu.SemaphoreType.DMA,
        pltpu.SemaphoreType.DMA,
        pltpu.SemaphoreType.DMA,
        pltpu.SemaphoreType.DMA,
    ],
)

_BM = 1024


def _tc_body(p_ref, w_ref, o_ref):
    o_ref[...] = jnp.dot(p_ref[0] + p_ref[1], w_ref[...],
                         preferred_element_type=jnp.float32)


def _tc_matmul(partials, weight):
    return pl.pallas_call(
        _tc_body,
        grid=(NP // _BM,),
        in_specs=[
            pl.BlockSpec((NC, _BM, D), lambda i: (0, i, 0)),
            pl.BlockSpec((D, D), lambda i: (0, 0)),
        ],
        out_specs=pl.BlockSpec((_BM, D), lambda i: (i, 0)),
        out_shape=jax.ShapeDtypeStruct((NP, D), jnp.float32),
    )(partials, weight)


@jax.jit
def _impl(x, edge_index, adj_values, weight):
    row = edge_index[0]
    col = edge_index[1]
    colp = jnp.zeros((E_PAD,), jnp.int32).at[:E].set(col)
    rowp = jnp.zeros((E_PAD,), jnp.int32).at[:E].set(row)
    adjp = jnp.zeros((E_PAD,), jnp.float32).at[:E].set(adj_values)
    colp = colp.reshape(NW, NSB, SCH, CHUNK)
    rowp = rowp.reshape(NW, NSB, SCH, CHUNK)
    adjp = adjp.reshape(NW, NSB, SCH * CHUNK)
    zeros = jnp.zeros((RPT, D), jnp.float32)
    partials = _sc_aggregate(x, colp, rowp, adjp, zeros)
    return _tc_matmul(partials, weight)[:N]


def kernel(x, edge_index, adj_values, weight):
    return _impl(x, edge_index, adj_values, weight)
